# baseline (speedup 1.0000x reference)
import functools

import jax
import jax.numpy as jnp
from jax import lax
from jax.experimental import pallas as pl
from jax.experimental.pallas import tpu as pltpu
from jax.experimental.pallas import tpu_sc as plsc

_B, _N, _DIM = 16, 1024, 16
_TOKENS = _B * _N
_VOCAB = 8192
_D = 2 * _DIM

_M_TILE = 1024
_M_STEPS = _TOKENS // _M_TILE

_SC_CORES = 2
_SC_SUBCORES = 16
_SC_LANES = 16
_SC_WORKERS = _SC_CORES * _SC_SUBCORES
_TOK_PER_WORKER = _TOKENS // _SC_WORKERS


def _bf16_split(x, n):
    parts = []
    rem = x
    for _ in range(n):
        h = rem.astype(jnp.bfloat16)
        parts.append(h)
        rem = rem - h.astype(jnp.float32)
    return parts


def _argmin_body(zr_ref, zi_ref, cb_ref, w_ref, b_ref,
                 idx_ref, salcode_ref, loss_ref, caug_ref):
    m = pl.program_id(0)

    @pl.when(m == 0)
    def _():
        c = cb_ref[...]
        cn = jnp.sum(c * c, axis=1, keepdims=True)
        cn_p = _bf16_split(cn, 3)
        caug_ref[...] = jnp.concatenate(
            [c.astype(jnp.bfloat16)] + cn_p, axis=1)
        salcode_ref[...] = lax.dot_general(
            w_ref[...], c, (((1,), (1,)), ((), ())),
            preferred_element_type=jnp.float32) + b_ref[0, 0]

    z2 = jnp.concatenate([zr_ref[...], zi_ref[...]], axis=1)
    ones = jnp.ones((_M_TILE, 1), dtype=jnp.bfloat16)
    z_aug = jnp.concatenate(
        [(-2.0 * z2).astype(jnp.bfloat16), ones, ones, ones], axis=1)
    d = lax.dot_general(z_aug, caug_ref[...], (((1,), (1,)), ((), ())),
                        preferred_element_type=jnp.float32)
    idx_ref[...] = jnp.argmin(d, axis=1).astype(jnp.int32).reshape(_M_TILE, 1)

    step_sum = (jnp.sum(jnp.min(d, axis=1), keepdims=True).reshape(1, 1)
                + jnp.sum(z2 * z2, keepdims=True).reshape(1, 1))

    @pl.when(m == 0)
    def _():
        loss_ref[...] = step_sum

    @pl.when(m > 0)
    def _():
        loss_ref[...] = loss_ref[...] + step_sum

    @pl.when(m == _M_STEPS - 1)
    def _():
        loss_ref[...] = loss_ref[...] * (1.25 / (_TOKENS * _D))


_argmin_call = pl.pallas_call(
    _argmin_body,
    grid=(_M_STEPS,),
    in_specs=[
        pl.BlockSpec((_M_TILE, _DIM), lambda m: (m, 0)),
        pl.BlockSpec((_M_TILE, _DIM), lambda m: (m, 0)),
        pl.BlockSpec((_VOCAB, _D), lambda m: (0, 0)),
        pl.BlockSpec((1, _D), lambda m: (0, 0)),
        pl.BlockSpec((1, 1), lambda m: (0, 0)),
    ],
    out_specs=(
        pl.BlockSpec((_M_TILE, 1), lambda m: (m, 0)),
        pl.BlockSpec((1, _VOCAB), lambda m: (0, 0)),
        pl.BlockSpec((1, 1), lambda m: (0, 0)),
    ),
    out_shape=(
        jax.ShapeDtypeStruct((_TOKENS, 1), jnp.int32),
        jax.ShapeDtypeStruct((1, _VOCAB), jnp.float32),
        jax.ShapeDtypeStruct((1, 1), jnp.float32),
    ),
    scratch_shapes=[
        pltpu.VMEM((_VOCAB, _D + 3), jnp.bfloat16),
    ],
)


@functools.cache
def _sc_gather_call():
    @functools.partial(
        pl.kernel,
        out_type=(
            jax.ShapeDtypeStruct((_TOKENS, _D), jnp.float32),
            jax.ShapeDtypeStruct((_TOKENS,), jnp.float32),
        ),
        mesh=plsc.VectorSubcoreMesh(core_axis_name="c", subcore_axis_name="s"),
        scratch_types=[
            pltpu.VMEM((_TOK_PER_WORKER,), jnp.int32),
            pltpu.VMEM((_TOK_PER_WORKER, _D), jnp.float32),
            pltpu.VMEM((_VOCAB,), jnp.float32),
            pltpu.VMEM((_TOK_PER_WORKER,), jnp.float32),
            pltpu.SemaphoreType.DMA,
        ],
        compiler_params=pltpu.CompilerParams(use_tc_tiling_on_sc=False,
                                             needs_layout_passes=False),
    )
    def _sc_gather(table_hbm, idx_hbm, salcode_hbm, rows_hbm, sal_hbm,
                   idx_v, rows_v, salcode_v, sal_v, sem):
        wid = lax.axis_index("s") * _SC_CORES + lax.axis_index("c")
        base = wid * _TOK_PER_WORKER
        pltpu.sync_copy(idx_hbm.at[pl.ds(base, _TOK_PER_WORKER)], idx_v)
        pltpu.async_copy(table_hbm.at[idx_v], rows_v, sem).wait()
        pltpu.sync_copy(rows_v, rows_hbm.at[pl.ds(base, _TOK_PER_WORKER)])
        pltpu.sync_copy(salcode_hbm, salcode_v)

        def body(i, carry):
            sl = pl.ds(i * _SC_LANES, _SC_LANES)
            sal_v[sl] = plsc.load_gather(salcode_v, [idx_v[sl]])
            return carry

        lax.fori_loop(0, _TOK_PER_WORKER // _SC_LANES, body, 0)
        pltpu.sync_copy(sal_v, sal_hbm.at[pl.ds(base, _TOK_PER_WORKER)])

    return _sc_gather


def kernel(gw_real, gw_imag, codebook, sal_w, sal_b):
    zr = gw_real.reshape(_TOKENS, _DIM)
    zi = gw_imag.reshape(_TOKENS, _DIM)
    idx, salcode, loss = _argmin_call(zr, zi, codebook, sal_w,
                                      sal_b.reshape(1, 1))
    z_q, sal = _sc_gather_call()(codebook, idx.reshape(_TOKENS), salcode.reshape(_VOCAB))
    proposal = lax.complex(z_q[:, :_DIM], z_q[:, _DIM:]).reshape(_B, _N, _DIM)
    salience = sal.reshape(_B, _N, 1)
    return proposal, salience, loss.reshape(())


# speedup vs baseline: 1.0125x; 1.0125x over previous
import functools

import jax
import jax.numpy as jnp
from jax import lax
from jax.experimental import pallas as pl
from jax.experimental.pallas import tpu as pltpu
from jax.experimental.pallas import tpu_sc as plsc

_B, _N, _DIM = 16, 1024, 16
_TOKENS = _B * _N
_VOCAB = 8192
_D = 2 * _DIM

_M_TILE = 512
_M_STEPS = _TOKENS // _M_TILE

_SC_CORES = 2
_SC_SUBCORES = 16
_SC_LANES = 16
_SC_WORKERS = _SC_CORES * _SC_SUBCORES
_TOK_PER_WORKER = _TOKENS // _SC_WORKERS


def _bf16_split(x, n):
    parts = []
    rem = x
    for _ in range(n):
        h = rem.astype(jnp.bfloat16)
        parts.append(h)
        rem = rem - h.astype(jnp.float32)
    return parts


def _argmin_body(zr_ref, zi_ref, cb_ref, w_ref, b_ref,
                 idx_ref, salcode_ref, loss_ref, caug_ref):
    m = pl.program_id(0)

    @pl.when(m == 0)
    def _():
        c = cb_ref[...]
        cn = jnp.sum(c * c, axis=1, keepdims=True)
        cn_p = _bf16_split(cn, 3)
        caug_ref[...] = jnp.concatenate(
            [c.astype(jnp.bfloat16)] + cn_p, axis=1)
        salcode_ref[...] = lax.dot_general(
            w_ref[...], c, (((1,), (1,)), ((), ())),
            preferred_element_type=jnp.float32) + b_ref[0, 0]

    z2 = jnp.concatenate([zr_ref[...], zi_ref[...]], axis=1)
    ones = jnp.ones((_M_TILE, 1), dtype=jnp.bfloat16)
    z_aug = jnp.concatenate(
        [(-2.0 * z2).astype(jnp.bfloat16), ones, ones, ones], axis=1)
    d = lax.dot_general(z_aug, caug_ref[...], (((1,), (1,)), ((), ())),
                        preferred_element_type=jnp.float32)
    idx_ref[...] = jnp.argmin(d, axis=1).astype(jnp.int32).reshape(_M_TILE, 1)

    step_sum = (jnp.sum(jnp.min(d, axis=1), keepdims=True).reshape(1, 1)
                + jnp.sum(z2 * z2, keepdims=True).reshape(1, 1))

    @pl.when(m == 0)
    def _():
        loss_ref[...] = step_sum

    @pl.when(m > 0)
    def _():
        loss_ref[...] = loss_ref[...] + step_sum

    @pl.when(m == _M_STEPS - 1)
    def _():
        loss_ref[...] = loss_ref[...] * (1.25 / (_TOKENS * _D))


_argmin_call = pl.pallas_call(
    _argmin_body,
    grid=(_M_STEPS,),
    in_specs=[
        pl.BlockSpec((_M_TILE, _DIM), lambda m: (m, 0)),
        pl.BlockSpec((_M_TILE, _DIM), lambda m: (m, 0)),
        pl.BlockSpec((_VOCAB, _D), lambda m: (0, 0)),
        pl.BlockSpec((1, _D), lambda m: (0, 0)),
        pl.BlockSpec((1, 1), lambda m: (0, 0)),
    ],
    out_specs=(
        pl.BlockSpec((_M_TILE, 1), lambda m: (m, 0)),
        pl.BlockSpec((1, _VOCAB), lambda m: (0, 0)),
        pl.BlockSpec((1, 1), lambda m: (0, 0)),
    ),
    out_shape=(
        jax.ShapeDtypeStruct((_TOKENS, 1), jnp.int32),
        jax.ShapeDtypeStruct((1, _VOCAB), jnp.float32),
        jax.ShapeDtypeStruct((1, 1), jnp.float32),
    ),
    scratch_shapes=[
        pltpu.VMEM((_VOCAB, _D + 3), jnp.bfloat16),
    ],
)


@functools.cache
def _sc_gather_call():
    @functools.partial(
        pl.kernel,
        out_type=(
            jax.ShapeDtypeStruct((_TOKENS, _D), jnp.float32),
            jax.ShapeDtypeStruct((_TOKENS,), jnp.float32),
        ),
        mesh=plsc.VectorSubcoreMesh(core_axis_name="c", subcore_axis_name="s"),
        scratch_types=[
            pltpu.VMEM((_TOK_PER_WORKER,), jnp.int32),
            pltpu.VMEM((_TOK_PER_WORKER, _D), jnp.float32),
            pltpu.VMEM((_VOCAB,), jnp.float32),
            pltpu.VMEM((_TOK_PER_WORKER,), jnp.float32),
            pltpu.SemaphoreType.DMA,
        ],
        compiler_params=pltpu.CompilerParams(use_tc_tiling_on_sc=False,
                                             needs_layout_passes=False),
    )
    def _sc_gather(table_hbm, idx_hbm, salcode_hbm, rows_hbm, sal_hbm,
                   idx_v, rows_v, salcode_v, sal_v, sem):
        wid = lax.axis_index("s") * _SC_CORES + lax.axis_index("c")
        base = wid * _TOK_PER_WORKER
        pltpu.sync_copy(idx_hbm.at[pl.ds(base, _TOK_PER_WORKER)], idx_v)
        pltpu.async_copy(table_hbm.at[idx_v], rows_v, sem).wait()
        pltpu.sync_copy(rows_v, rows_hbm.at[pl.ds(base, _TOK_PER_WORKER)])
        pltpu.sync_copy(salcode_hbm, salcode_v)

        def body(i, carry):
            sl = pl.ds(i * _SC_LANES, _SC_LANES)
            sal_v[sl] = plsc.load_gather(salcode_v, [idx_v[sl]])
            return carry

        lax.fori_loop(0, _TOK_PER_WORKER // _SC_LANES, body, 0)
        pltpu.sync_copy(sal_v, sal_hbm.at[pl.ds(base, _TOK_PER_WORKER)])

    return _sc_gather


def kernel(gw_real, gw_imag, codebook, sal_w, sal_b):
    zr = gw_real.reshape(_TOKENS, _DIM)
    zi = gw_imag.reshape(_TOKENS, _DIM)
    idx, salcode, loss = _argmin_call(zr, zi, codebook, sal_w,
                                      sal_b.reshape(1, 1))
    z_q, sal = _sc_gather_call()(codebook, idx.reshape(_TOKENS), salcode.reshape(_VOCAB))
    proposal = lax.complex(z_q[:, :_DIM], z_q[:, _DIM:]).reshape(_B, _N, _DIM)
    salience = sal.reshape(_B, _N, 1)
    return proposal, salience, loss.reshape(())


# speedup vs baseline: 1.0324x; 1.0196x over previous
import functools

import jax
import jax.numpy as jnp
from jax import lax
from jax.experimental import pallas as pl
from jax.experimental.pallas import tpu as pltpu
from jax.experimental.pallas import tpu_sc as plsc

_B, _N, _DIM = 16, 1024, 16
_TOKENS = _B * _N
_VOCAB = 8192
_D = 2 * _DIM

_M_TILE = 512
_M_STEPS = _TOKENS // _M_TILE

_SC_CORES = 2
_SC_SUBCORES = 16
_SC_LANES = 16
_SC_WORKERS = _SC_CORES * _SC_SUBCORES
_TOK_PER_WORKER = _TOKENS // _SC_WORKERS


def _bf16_split(x, n):
    parts = []
    rem = x
    for _ in range(n):
        h = rem.astype(jnp.bfloat16)
        parts.append(h)
        rem = rem - h.astype(jnp.float32)
    return parts


def _argmin_body(zr_ref, zi_ref, cb_ref, w_ref, b_ref,
                 idx_ref, salcode_ref, loss_ref, caug_ref):
    m = pl.program_id(0)

    @pl.when(m == 0)
    def _():
        c = cb_ref[...]
        cn = jnp.sum(c * c, axis=1, keepdims=True)
        cn_p = _bf16_split(cn, 3)
        caug_ref[...] = jnp.concatenate(
            [c.astype(jnp.bfloat16)] + cn_p, axis=1)
        salcode_ref[...] = lax.dot_general(
            w_ref[...], c, (((1,), (1,)), ((), ())),
            preferred_element_type=jnp.float32) + b_ref[0, 0]

    z2 = jnp.concatenate([zr_ref[...], zi_ref[...]], axis=1)
    ones = jnp.ones((_M_TILE, 1), dtype=jnp.bfloat16)
    z_aug = jnp.concatenate(
        [(-2.0 * z2).astype(jnp.bfloat16), ones, ones, ones], axis=1)
    d = lax.dot_general(z_aug, caug_ref[...], (((1,), (1,)), ((), ())),
                        preferred_element_type=jnp.float32)
    idx_ref[...] = jnp.argmin(d, axis=1).astype(jnp.int32)

    step_sum = (jnp.sum(jnp.min(d, axis=1), keepdims=True).reshape(1, 1)
                + jnp.sum(z2 * z2, keepdims=True).reshape(1, 1))

    @pl.when(m == 0)
    def _():
        loss_ref[...] = step_sum

    @pl.when(m > 0)
    def _():
        loss_ref[...] = loss_ref[...] + step_sum

    @pl.when(m == _M_STEPS - 1)
    def _():
        loss_ref[...] = loss_ref[...] * (1.25 / (_TOKENS * _D))


_argmin_call = pl.pallas_call(
    _argmin_body,
    grid=(_M_STEPS,),
    in_specs=[
        pl.BlockSpec((_M_TILE, _DIM), lambda m: (m, 0)),
        pl.BlockSpec((_M_TILE, _DIM), lambda m: (m, 0)),
        pl.BlockSpec((_VOCAB, _D), lambda m: (0, 0)),
        pl.BlockSpec((1, _D), lambda m: (0, 0)),
        pl.BlockSpec((1, 1), lambda m: (0, 0)),
    ],
    out_specs=(
        pl.BlockSpec((_M_TILE,), lambda m: (m,)),
        pl.BlockSpec((1, _VOCAB), lambda m: (0, 0)),
        pl.BlockSpec((1, 1), lambda m: (0, 0)),
    ),
    out_shape=(
        jax.ShapeDtypeStruct((_TOKENS,), jnp.int32),
        jax.ShapeDtypeStruct((1, _VOCAB), jnp.float32),
        jax.ShapeDtypeStruct((1, 1), jnp.float32),
    ),
    scratch_shapes=[
        pltpu.VMEM((_VOCAB, _D + 3), jnp.bfloat16),
    ],
)


@functools.cache
def _sc_gather_call():
    @functools.partial(
        pl.kernel,
        out_type=(
            jax.ShapeDtypeStruct((_TOKENS, _D), jnp.float32),
            jax.ShapeDtypeStruct((_TOKENS,), jnp.float32),
        ),
        mesh=plsc.VectorSubcoreMesh(core_axis_name="c", subcore_axis_name="s"),
        scratch_types=[
            pltpu.VMEM((_TOK_PER_WORKER,), jnp.int32),
            pltpu.VMEM((_TOK_PER_WORKER, _D), jnp.float32),
            pltpu.VMEM((_VOCAB,), jnp.float32),
            pltpu.VMEM((_TOK_PER_WORKER,), jnp.float32),
            pltpu.SemaphoreType.DMA,
        ],
        compiler_params=pltpu.CompilerParams(use_tc_tiling_on_sc=False,
                                             needs_layout_passes=False),
    )
    def _sc_gather(table_hbm, idx_hbm, salcode_hbm, rows_hbm, sal_hbm,
                   idx_v, rows_v, salcode_v, sal_v, sem):
        wid = lax.axis_index("s") * _SC_CORES + lax.axis_index("c")
        base = wid * _TOK_PER_WORKER
        pltpu.sync_copy(idx_hbm.at[pl.ds(base, _TOK_PER_WORKER)], idx_v)
        pltpu.async_copy(table_hbm.at[idx_v], rows_v, sem).wait()
        pltpu.sync_copy(rows_v, rows_hbm.at[pl.ds(base, _TOK_PER_WORKER)])
        pltpu.sync_copy(salcode_hbm, salcode_v)

        def body(i, carry):
            sl = pl.ds(i * _SC_LANES, _SC_LANES)
            sal_v[sl] = plsc.load_gather(salcode_v, [idx_v[sl]])
            return carry

        lax.fori_loop(0, _TOK_PER_WORKER // _SC_LANES, body, 0)
        pltpu.sync_copy(sal_v, sal_hbm.at[pl.ds(base, _TOK_PER_WORKER)])

    return _sc_gather


def kernel(gw_real, gw_imag, codebook, sal_w, sal_b):
    zr = gw_real.reshape(_TOKENS, _DIM)
    zi = gw_imag.reshape(_TOKENS, _DIM)
    idx, salcode, loss = _argmin_call(zr, zi, codebook, sal_w,
                                      sal_b.reshape(1, 1))
    z_q, sal = _sc_gather_call()(codebook, idx, salcode.reshape(_VOCAB))
    proposal = lax.complex(z_q[:, :_DIM], z_q[:, _DIM:]).reshape(_B, _N, _DIM)
    salience = sal.reshape(_B, _N, 1)
    return proposal, salience, loss.reshape(())
